# baseline (speedup 1.0000x reference)
import functools

import numpy as np

import jax
import jax.numpy as jnp
from jax import lax
from jax.experimental import pallas as pl
from jax.experimental.pallas import tpu as pltpu
from jax.experimental.pallas import tpu_sc as plsc

NN = 10000
EE = 320000
DD = 128
HH = 128
LL = 64
BB = 64
MM = 100
TRI = MM * (MM - 1) // 2
TRIP = 4992

NP_ = 10240
RPT = NP_ // 16
CH = 128
EP = 327680
NCH = EP // (32 * CH)
PC = 8
FAST_P = 13
SLOW_P = 7
FAST_C = 1
DW = 128
ADJP = 10240
NTRI = 9904
RB = 512

_mesh = plsc.VectorSubcoreMesh(core_axis_name="c", subcore_axis_name="s")


@functools.partial(
    pl.kernel,
    out_type=jax.ShapeDtypeStruct((2 * NP_, DW), jnp.float32),
    mesh=_mesh,
    scratch_types=[
        pltpu.VMEM((NCH, CH), jnp.int32),
        pltpu.VMEM((CH, DW), jnp.float32),
        pltpu.VMEM_SHARED((NP_, DW), jnp.float32),
    ],
)
def _deg_sc(dst_hbm, ones_hbm, zeros_hbm, out_hbm, idx_v, ones_v, acc):
    c = lax.axis_index("c")
    s = lax.axis_index("s")
    wid = s * 2 + c
    pltpu.sync_copy(dst_hbm.at[pl.ds(wid * NCH, NCH)], idx_v)
    pltpu.sync_copy(ones_hbm, ones_v)
    pltpu.sync_copy(zeros_hbm, acc.at[pl.ds(s * RPT, RPT)])
    plsc.subcore_barrier()

    def body(j, carry):
        pltpu.sync_copy(ones_v, acc.at[idx_v.at[j]], add=True)
        return carry

    lax.fori_loop(0, NCH, body, 0)
    plsc.subcore_barrier()
    pltpu.sync_copy(acc.at[pl.ds(s * RPT, RPT)],
                    out_hbm.at[pl.ds(c * NP_ + s * RPT, RPT)])


@functools.partial(
    pl.kernel,
    out_type=jax.ShapeDtypeStruct((2 * NP_, HH), jnp.float32),
    mesh=_mesh,
    scratch_types=[
        pltpu.VMEM((PC, CH), jnp.int32),
        pltpu.VMEM((PC, CH), jnp.int32),
        pltpu.VMEM((CH, HH), jnp.float32),
        pltpu.VMEM((CH, HH), jnp.float32),
        pltpu.VMEM_SHARED((NP_, HH), jnp.float32),
        pltpu.SemaphoreType.DMA,
        pltpu.SemaphoreType.DMA,
    ],
)
def _msg_sc(g_hbm, src_hbm, dst_hbm, zeros_hbm, out_hbm, sidx, didx,
            b0, b1, acc, s0, s1):
    c = lax.axis_index("c")
    s = lax.axis_index("s")
    bufs = (b0, b1)
    sems = (s0, s1)
    pltpu.sync_copy(zeros_hbm, acc.at[pl.ds(s * RPT, RPT)])
    plsc.subcore_barrier()

    def do_piece(base):
        pltpu.sync_copy(src_hbm.at[pl.ds(base, PC)], sidx)
        pltpu.sync_copy(dst_hbm.at[pl.ds(base, PC)], didx)
        pltpu.async_copy(g_hbm.at[sidx.at[0]], bufs[0], sems[0])

        def body(j2, carry):
            for k in range(2):
                j = j2 * 2 + k
                pltpu.make_async_copy(g_hbm.at[sidx.at[j]], bufs[k],
                                      sems[k]).wait()
                jn = j + 1

                @pl.when(jn < PC)
                def _():
                    pltpu.async_copy(g_hbm.at[sidx.at[jn]], bufs[(k + 1) % 2],
                                     sems[(k + 1) % 2])

                pltpu.sync_copy(bufs[k], acc.at[didx.at[j]], add=True)
            return carry

        lax.fori_loop(0, PC // 2, body, 0)

    @pl.when(c == FAST_C)
    def _():
        for p in range(FAST_P):
            do_piece((s * FAST_P + p) * PC)

    @pl.when(c != FAST_C)
    def _():
        for p in range(SLOW_P):
            do_piece(16 * FAST_P * PC + (s * SLOW_P + p) * PC)

    plsc.subcore_barrier()
    pltpu.sync_copy(acc.at[pl.ds(s * RPT, RPT)],
                    out_hbm.at[pl.ds(c * NP_ + s * RPT, RPT)])


@functools.partial(
    pl.kernel,
    out_type=jax.ShapeDtypeStruct((BB * ADJP,), jnp.float32),
    mesh=_mesh,
    scratch_types=[
        pltpu.VMEM((NTRI,), jnp.int32),
        pltpu.VMEM((NTRI,), jnp.int32),
        pltpu.VMEM((TRIP,), jnp.float32),
        pltpu.VMEM((ADJP,), jnp.float32),
    ],
    compiler_params=pltpu.CompilerParams(needs_layout_passes=False),
)
def _triu_sc(probs_hbm, pos_hbm, vsrc_hbm, zrow_hbm, out_hbm, pos_v, vsrc_v, prow, row):
    c = lax.axis_index("c")
    s = lax.axis_index("s")
    wid = s * 2 + c
    pltpu.sync_copy(pos_hbm, pos_v)
    pltpu.sync_copy(vsrc_hbm, vsrc_v)
    for gi in range(2):
        b = wid * 2 + gi
        pltpu.sync_copy(zrow_hbm, row)
        pltpu.sync_copy(probs_hbm.at[pl.ds(b * TRIP, TRIP)], prow)

        def body(t, carry):
            i16 = pos_v[pl.ds(t * 16, 16)]
            s16 = vsrc_v[pl.ds(t * 16, 16)]
            v = plsc.load_gather(prow, [s16])
            plsc.store_scatter(row, [i16], v)
            return carry

        lax.fori_loop(0, NTRI // 16, body, 0)
        pltpu.sync_copy(row, out_hbm.at[pl.ds(b * ADJP, ADJP)])


def _gmat_body(x_ref, w_ref, d0_ref, d1_ref, g_ref):
    dinv = lax.rsqrt(d0_ref[...] + d1_ref[...] + 1.0)
    g_ref[...] = jnp.dot(x_ref[...], w_ref[...],
                         preferred_element_type=jnp.float32) * dinv


_gmat = pl.pallas_call(
    _gmat_body,
    grid=(NP_ // RB,),
    in_specs=[
        pl.BlockSpec((RB, DD), lambda i: (i, 0)),
        pl.BlockSpec((DD, HH), lambda i: (0, 0)),
        pl.BlockSpec((RB, 1), lambda i: (i, 0)),
        pl.BlockSpec((RB, 1), lambda i: (i, 0)),
    ],
    out_specs=pl.BlockSpec((RB, HH), lambda i: (i, 0)),
    out_shape=jax.ShapeDtypeStruct((NP_, HH), jnp.float32),
)


def _cb2_body(p0_ref, p1_ref, g_ref, d0_ref, d1_ref, b_ref, w_ref, o_ref):
    i = pl.program_id(0)
    dinv = lax.rsqrt(d0_ref[...] + d1_ref[...] + 1.0)
    h = (p0_ref[...] + p1_ref[...] + g_ref[...]) * dinv + b_ref[...]
    h = jnp.maximum(h, 0.0)
    rows = i * RB + lax.broadcasted_iota(jnp.int32, (RB, 1), 0)
    h = jnp.where(rows < NN, h, 0.0)
    o_ref[...] = jnp.dot(h, w_ref[...],
                         preferred_element_type=jnp.float32) * dinv


_cb2 = pl.pallas_call(
    _cb2_body,
    grid=(NP_ // RB,),
    in_specs=[
        pl.BlockSpec((RB, HH), lambda i: (i, 0)),
        pl.BlockSpec((RB, HH), lambda i: (i, 0)),
        pl.BlockSpec((RB, HH), lambda i: (i, 0)),
        pl.BlockSpec((RB, 1), lambda i: (i, 0)),
        pl.BlockSpec((RB, 1), lambda i: (i, 0)),
        pl.BlockSpec((1, HH), lambda i: (0, 0)),
        pl.BlockSpec((HH, HH), lambda i: (0, 0)),
    ],
    out_specs=pl.BlockSpec((RB, HH), lambda i: (i, 0)),
    out_shape=jax.ShapeDtypeStruct((NP_, HH), jnp.float32),
)


def _pool_body(p0_ref, p1_ref, g_ref, d0_ref, d1_ref, b_ref, bat_ref,
               sums_ref, cnts_ref):
    i = pl.program_id(0)
    dinv = lax.rsqrt(d0_ref[...] + d1_ref[...] + 1.0)
    h = (p0_ref[...] + p1_ref[...] + g_ref[...]) * dinv + b_ref[...]
    h = jnp.maximum(h, 0.0)
    oh = (bat_ref[...] == lax.broadcasted_iota(jnp.int32, (RB, BB), 1))
    oh = oh.astype(jnp.float32)
    ps = lax.dot_general(oh, h, (((0,), (0,)), ((), ())),
                         preferred_element_type=jnp.float32)
    pc = lax.dot_general(oh, jnp.ones((RB, HH), jnp.float32),
                         (((0,), (0,)), ((), ())),
                         preferred_element_type=jnp.float32)

    @pl.when(i == 0)
    def _():
        sums_ref[...] = jnp.zeros_like(sums_ref)
        cnts_ref[...] = jnp.zeros_like(cnts_ref)

    sums_ref[...] += ps
    cnts_ref[...] += pc


_pool = pl.pallas_call(
    _pool_body,
    grid=(NP_ // RB,),
    in_specs=[
        pl.BlockSpec((RB, HH), lambda i: (i, 0)),
        pl.BlockSpec((RB, HH), lambda i: (i, 0)),
        pl.BlockSpec((RB, HH), lambda i: (i, 0)),
        pl.BlockSpec((RB, 1), lambda i: (i, 0)),
        pl.BlockSpec((RB, 1), lambda i: (i, 0)),
        pl.BlockSpec((1, HH), lambda i: (0, 0)),
        pl.BlockSpec((RB, 1), lambda i: (i, 0)),
    ],
    out_specs=[
        pl.BlockSpec((BB, HH), lambda i: (0, 0)),
        pl.BlockSpec((BB, HH), lambda i: (0, 0)),
    ],
    out_shape=[
        jax.ShapeDtypeStruct((BB, HH), jnp.float32),
        jax.ShapeDtypeStruct((BB, HH), jnp.float32),
    ],
)


def _head_body(sums_ref, cnts_ref, muW_ref, mub_ref, lvW_ref, lvb_ref, eps_ref,
               dW1_ref, db1_ref, dW2_ref, db2_ref, dW3_ref, db3_ref,
               mu_ref, lv_ref, probs_ref):
    hg = sums_ref[...] / jnp.maximum(cnts_ref[...], 1.0)
    mu = jnp.dot(hg, muW_ref[...], preferred_element_type=jnp.float32) + mub_ref[...]
    lv = jnp.dot(hg, lvW_ref[...], preferred_element_type=jnp.float32) + lvb_ref[...]
    mu_ref[...] = mu
    lv_ref[...] = lv
    z = mu + eps_ref[...] * jnp.exp(0.5 * lv)
    t1 = jnp.maximum(jnp.dot(z, dW1_ref[...],
                             preferred_element_type=jnp.float32) + db1_ref[...], 0.0)
    t2 = jnp.maximum(jnp.dot(t1, dW2_ref[...],
                             preferred_element_type=jnp.float32) + db2_ref[...], 0.0)
    logits = jnp.dot(t2, dW3_ref[...],
                     preferred_element_type=jnp.float32) + db3_ref[...]
    probs_ref[...] = jax.nn.sigmoid(logits)


_head = pl.pallas_call(
    _head_body,
    out_shape=[
        jax.ShapeDtypeStruct((BB, LL), jnp.float32),
        jax.ShapeDtypeStruct((BB, LL), jnp.float32),
        jax.ShapeDtypeStruct((BB, TRIP), jnp.float32),
    ],
)


def kernel(x, edge_index, batch, W1, b1, W2, b2, muW, mub, lvW, lvb,
           dW1, db1, dW2, db2, dW3, db3):
    f32 = jnp.float32
    src = edge_index[0].astype(jnp.int32)
    dst = edge_index[1].astype(jnp.int32)
    padi = jnp.full((EP - EE,), NN, jnp.int32)
    srcp = jnp.concatenate([src, padi]).reshape(EP // CH, CH)
    dstp = jnp.concatenate([dst, padi]).reshape(EP // CH, CH)
    xp = jnp.pad(x, ((0, NP_ - NN), (0, 0)))
    batp = jnp.pad(batch.astype(jnp.int32), (0, NP_ - NN),
                   constant_values=BB).reshape(NP_, 1)
    onesd = jnp.ones((CH, DW), f32)
    zerosd = jnp.zeros((RPT, DW), f32)
    zerosh = jnp.zeros((RPT, HH), f32)
    zrow = jnp.zeros((ADJP,), f32)
    eps = jax.random.normal(jax.random.key(1), (BB, LL), dtype=f32)

    iu, ju = np.triu_indices(MM, k=1)
    pos_np = np.concatenate([iu * MM + ju, ju * MM + iu]).astype(np.int32)
    pos_np = np.pad(pos_np, (0, NTRI - 2 * TRI), constant_values=MM * MM + 8)
    vsrc_np = np.concatenate([np.arange(TRI), np.arange(TRI)]).astype(np.int32)
    vsrc_np = np.pad(vsrc_np, (0, NTRI - 2 * TRI))
    pos_j = jnp.asarray(pos_np)
    vsrc_j = jnp.asarray(vsrc_np)

    dW3p = jnp.pad(dW3, ((0, 0), (0, TRIP - TRI)))
    db3p = jnp.pad(db3, (0, TRIP - TRI)).reshape(1, TRIP)

    degf = _deg_sc(dstp, onesd, zerosd)
    d0 = degf[:NP_, 0:1]
    d1 = degf[NP_:, 0:1]

    g1 = _gmat(xp, W1, d0, d1)
    parts1 = _msg_sc(g1, srcp, dstp, zerosh)
    g2 = _cb2(parts1[:NP_], parts1[NP_:], g1, d0, d1, b1.reshape(1, HH), W2)
    parts2 = _msg_sc(g2, srcp, dstp, zerosh)
    sums, cnts = _pool(parts2[:NP_], parts2[NP_:], g2, d0, d1,
                       b2.reshape(1, HH), batp)
    mu, lv, probsp = _head(sums, cnts, muW, mub.reshape(1, LL), lvW,
                           lvb.reshape(1, LL), eps, dW1, db1.reshape(1, HH),
                           dW2, db2.reshape(1, HH), dW3p, db3p)
    adjf = _triu_sc(probsp.reshape(BB * TRIP), pos_j, vsrc_j, zrow)
    adj = adjf.reshape(BB, ADJP)[:, :MM * MM].reshape(BB, MM, MM)
    return (adj, mu, lv)


# speedup vs baseline: 1.0222x; 1.0222x over previous
import functools

import numpy as np

import jax
import jax.numpy as jnp
from jax import lax
from jax.experimental import pallas as pl
from jax.experimental.pallas import tpu as pltpu
from jax.experimental.pallas import tpu_sc as plsc

NN = 10000
EE = 320000
DD = 128
HH = 128
LL = 64
BB = 64
MM = 100
TRI = MM * (MM - 1) // 2
TRIP = 4992

NP_ = 10240
RPT = NP_ // 16
CH = 128
EP = 327680
NCH = EP // (32 * CH)
PC = 16
FAST_P = 7
SLOW_P = 3
FAST_C = 1
DW = 128
ADJP = 10240
NTRI = 9904
RB = 512

_mesh = plsc.VectorSubcoreMesh(core_axis_name="c", subcore_axis_name="s")


@functools.partial(
    pl.kernel,
    out_type=jax.ShapeDtypeStruct((2 * NP_, DW), jnp.float32),
    mesh=_mesh,
    scratch_types=[
        pltpu.VMEM((NCH, CH), jnp.int32),
        pltpu.VMEM((CH, DW), jnp.float32),
        pltpu.VMEM_SHARED((NP_, DW), jnp.float32),
    ],
)
def _deg_sc(dst_hbm, ones_hbm, zeros_hbm, out_hbm, idx_v, ones_v, acc):
    c = lax.axis_index("c")
    s = lax.axis_index("s")
    wid = s * 2 + c
    pltpu.sync_copy(dst_hbm.at[pl.ds(wid * NCH, NCH)], idx_v)
    pltpu.sync_copy(ones_hbm, ones_v)
    pltpu.sync_copy(zeros_hbm, acc.at[pl.ds(s * RPT, RPT)])
    plsc.subcore_barrier()

    def body(j, carry):
        pltpu.sync_copy(ones_v, acc.at[idx_v.at[j]], add=True)
        return carry

    lax.fori_loop(0, NCH, body, 0)
    plsc.subcore_barrier()
    pltpu.sync_copy(acc.at[pl.ds(s * RPT, RPT)],
                    out_hbm.at[pl.ds(c * NP_ + s * RPT, RPT)])


@functools.partial(
    pl.kernel,
    out_type=jax.ShapeDtypeStruct((2 * NP_, HH), jnp.float32),
    mesh=_mesh,
    scratch_types=[
        pltpu.VMEM((PC, CH), jnp.int32),
        pltpu.VMEM((PC, CH), jnp.int32),
        pltpu.VMEM((CH, HH), jnp.float32),
        pltpu.VMEM((CH, HH), jnp.float32),
        pltpu.VMEM_SHARED((NP_, HH), jnp.float32),
        pltpu.SemaphoreType.DMA,
        pltpu.SemaphoreType.DMA,
    ],
)
def _msg_sc(g_hbm, src_hbm, dst_hbm, zeros_hbm, out_hbm, sidx, didx,
            b0, b1, acc, s0, s1):
    c = lax.axis_index("c")
    s = lax.axis_index("s")
    bufs = (b0, b1)
    sems = (s0, s1)
    pltpu.sync_copy(zeros_hbm, acc.at[pl.ds(s * RPT, RPT)])
    plsc.subcore_barrier()

    def do_piece(base):
        pltpu.sync_copy(src_hbm.at[pl.ds(base, PC)], sidx)
        pltpu.sync_copy(dst_hbm.at[pl.ds(base, PC)], didx)
        pltpu.async_copy(g_hbm.at[sidx.at[0]], bufs[0], sems[0])

        def body(j2, carry):
            for k in range(2):
                j = j2 * 2 + k
                pltpu.make_async_copy(g_hbm.at[sidx.at[j]], bufs[k],
                                      sems[k]).wait()
                jn = j + 1

                @pl.when(jn < PC)
                def _():
                    pltpu.async_copy(g_hbm.at[sidx.at[jn]], bufs[(k + 1) % 2],
                                     sems[(k + 1) % 2])

                pltpu.sync_copy(bufs[k], acc.at[didx.at[j]], add=True)
            return carry

        lax.fori_loop(0, PC // 2, body, 0)

    @pl.when(c == FAST_C)
    def _():
        for p in range(FAST_P):
            do_piece((s * FAST_P + p) * PC)

    @pl.when(c != FAST_C)
    def _():
        for p in range(SLOW_P):
            do_piece(16 * FAST_P * PC + (s * SLOW_P + p) * PC)

    plsc.subcore_barrier()
    pltpu.sync_copy(acc.at[pl.ds(s * RPT, RPT)],
                    out_hbm.at[pl.ds(c * NP_ + s * RPT, RPT)])


@functools.partial(
    pl.kernel,
    out_type=jax.ShapeDtypeStruct((BB * ADJP,), jnp.float32),
    mesh=_mesh,
    scratch_types=[
        pltpu.VMEM((NTRI,), jnp.int32),
        pltpu.VMEM((NTRI,), jnp.int32),
        pltpu.VMEM((TRIP,), jnp.float32),
        pltpu.VMEM((ADJP,), jnp.float32),
    ],
    compiler_params=pltpu.CompilerParams(needs_layout_passes=False),
)
def _triu_sc(probs_hbm, pos_hbm, vsrc_hbm, zrow_hbm, out_hbm, pos_v, vsrc_v, prow, row):
    c = lax.axis_index("c")
    s = lax.axis_index("s")
    wid = s * 2 + c
    pltpu.sync_copy(pos_hbm, pos_v)
    pltpu.sync_copy(vsrc_hbm, vsrc_v)
    for gi in range(2):
        b = wid * 2 + gi
        pltpu.sync_copy(zrow_hbm, row)
        pltpu.sync_copy(probs_hbm.at[pl.ds(b * TRIP, TRIP)], prow)

        def body(t, carry):
            i16 = pos_v[pl.ds(t * 16, 16)]
            s16 = vsrc_v[pl.ds(t * 16, 16)]
            v = plsc.load_gather(prow, [s16])
            plsc.store_scatter(row, [i16], v)
            return carry

        lax.fori_loop(0, NTRI // 16, body, 0)
        pltpu.sync_copy(row, out_hbm.at[pl.ds(b * ADJP, ADJP)])


def _gmat_body(x_ref, w_ref, d0_ref, d1_ref, g_ref):
    dinv = lax.rsqrt(d0_ref[...] + d1_ref[...] + 1.0)
    g_ref[...] = jnp.dot(x_ref[...], w_ref[...],
                         preferred_element_type=jnp.float32) * dinv


_gmat = pl.pallas_call(
    _gmat_body,
    grid=(NP_ // RB,),
    in_specs=[
        pl.BlockSpec((RB, DD), lambda i: (i, 0)),
        pl.BlockSpec((DD, HH), lambda i: (0, 0)),
        pl.BlockSpec((RB, 1), lambda i: (i, 0)),
        pl.BlockSpec((RB, 1), lambda i: (i, 0)),
    ],
    out_specs=pl.BlockSpec((RB, HH), lambda i: (i, 0)),
    out_shape=jax.ShapeDtypeStruct((NP_, HH), jnp.float32),
)


def _cb2_body(p0_ref, p1_ref, g_ref, d0_ref, d1_ref, b_ref, w_ref, o_ref):
    i = pl.program_id(0)
    dinv = lax.rsqrt(d0_ref[...] + d1_ref[...] + 1.0)
    h = (p0_ref[...] + p1_ref[...] + g_ref[...]) * dinv + b_ref[...]
    h = jnp.maximum(h, 0.0)
    rows = i * RB + lax.broadcasted_iota(jnp.int32, (RB, 1), 0)
    h = jnp.where(rows < NN, h, 0.0)
    o_ref[...] = jnp.dot(h, w_ref[...],
                         preferred_element_type=jnp.float32) * dinv


_cb2 = pl.pallas_call(
    _cb2_body,
    grid=(NP_ // RB,),
    in_specs=[
        pl.BlockSpec((RB, HH), lambda i: (i, 0)),
        pl.BlockSpec((RB, HH), lambda i: (i, 0)),
        pl.BlockSpec((RB, HH), lambda i: (i, 0)),
        pl.BlockSpec((RB, 1), lambda i: (i, 0)),
        pl.BlockSpec((RB, 1), lambda i: (i, 0)),
        pl.BlockSpec((1, HH), lambda i: (0, 0)),
        pl.BlockSpec((HH, HH), lambda i: (0, 0)),
    ],
    out_specs=pl.BlockSpec((RB, HH), lambda i: (i, 0)),
    out_shape=jax.ShapeDtypeStruct((NP_, HH), jnp.float32),
)


def _pool_body(p0_ref, p1_ref, g_ref, d0_ref, d1_ref, b_ref, bat_ref,
               sums_ref, cnts_ref):
    i = pl.program_id(0)
    dinv = lax.rsqrt(d0_ref[...] + d1_ref[...] + 1.0)
    h = (p0_ref[...] + p1_ref[...] + g_ref[...]) * dinv + b_ref[...]
    h = jnp.maximum(h, 0.0)
    oh = (bat_ref[...] == lax.broadcasted_iota(jnp.int32, (RB, BB), 1))
    oh = oh.astype(jnp.float32)
    ps = lax.dot_general(oh, h, (((0,), (0,)), ((), ())),
                         preferred_element_type=jnp.float32)
    pc = lax.dot_general(oh, jnp.ones((RB, HH), jnp.float32),
                         (((0,), (0,)), ((), ())),
                         preferred_element_type=jnp.float32)

    @pl.when(i == 0)
    def _():
        sums_ref[...] = jnp.zeros_like(sums_ref)
        cnts_ref[...] = jnp.zeros_like(cnts_ref)

    sums_ref[...] += ps
    cnts_ref[...] += pc


_pool = pl.pallas_call(
    _pool_body,
    grid=(NP_ // RB,),
    in_specs=[
        pl.BlockSpec((RB, HH), lambda i: (i, 0)),
        pl.BlockSpec((RB, HH), lambda i: (i, 0)),
        pl.BlockSpec((RB, HH), lambda i: (i, 0)),
        pl.BlockSpec((RB, 1), lambda i: (i, 0)),
        pl.BlockSpec((RB, 1), lambda i: (i, 0)),
        pl.BlockSpec((1, HH), lambda i: (0, 0)),
        pl.BlockSpec((RB, 1), lambda i: (i, 0)),
    ],
    out_specs=[
        pl.BlockSpec((BB, HH), lambda i: (0, 0)),
        pl.BlockSpec((BB, HH), lambda i: (0, 0)),
    ],
    out_shape=[
        jax.ShapeDtypeStruct((BB, HH), jnp.float32),
        jax.ShapeDtypeStruct((BB, HH), jnp.float32),
    ],
)


def _head_body(sums_ref, cnts_ref, muW_ref, mub_ref, lvW_ref, lvb_ref, eps_ref,
               dW1_ref, db1_ref, dW2_ref, db2_ref, dW3_ref, db3_ref,
               mu_ref, lv_ref, probs_ref):
    hg = sums_ref[...] / jnp.maximum(cnts_ref[...], 1.0)
    mu = jnp.dot(hg, muW_ref[...], preferred_element_type=jnp.float32) + mub_ref[...]
    lv = jnp.dot(hg, lvW_ref[...], preferred_element_type=jnp.float32) + lvb_ref[...]
    mu_ref[...] = mu
    lv_ref[...] = lv
    z = mu + eps_ref[...] * jnp.exp(0.5 * lv)
    t1 = jnp.maximum(jnp.dot(z, dW1_ref[...],
                             preferred_element_type=jnp.float32) + db1_ref[...], 0.0)
    t2 = jnp.maximum(jnp.dot(t1, dW2_ref[...],
                             preferred_element_type=jnp.float32) + db2_ref[...], 0.0)
    logits = jnp.dot(t2, dW3_ref[...],
                     preferred_element_type=jnp.float32) + db3_ref[...]
    probs_ref[...] = jax.nn.sigmoid(logits)


_head = pl.pallas_call(
    _head_body,
    out_shape=[
        jax.ShapeDtypeStruct((BB, LL), jnp.float32),
        jax.ShapeDtypeStruct((BB, LL), jnp.float32),
        jax.ShapeDtypeStruct((BB, TRIP), jnp.float32),
    ],
)


def kernel(x, edge_index, batch, W1, b1, W2, b2, muW, mub, lvW, lvb,
           dW1, db1, dW2, db2, dW3, db3):
    f32 = jnp.float32
    src = edge_index[0].astype(jnp.int32)
    dst = edge_index[1].astype(jnp.int32)
    padi = jnp.full((EP - EE,), NN, jnp.int32)
    srcp = jnp.concatenate([src, padi]).reshape(EP // CH, CH)
    dstp = jnp.concatenate([dst, padi]).reshape(EP // CH, CH)
    xp = jnp.pad(x, ((0, NP_ - NN), (0, 0)))
    batp = jnp.pad(batch.astype(jnp.int32), (0, NP_ - NN),
                   constant_values=BB).reshape(NP_, 1)
    onesd = jnp.ones((CH, DW), f32)
    zerosd = jnp.zeros((RPT, DW), f32)
    zerosh = jnp.zeros((RPT, HH), f32)
    zrow = jnp.zeros((ADJP,), f32)
    eps = jax.random.normal(jax.random.key(1), (BB, LL), dtype=f32)

    iu, ju = np.triu_indices(MM, k=1)
    pos_np = np.concatenate([iu * MM + ju, ju * MM + iu]).astype(np.int32)
    pos_np = np.pad(pos_np, (0, NTRI - 2 * TRI), constant_values=MM * MM + 8)
    vsrc_np = np.concatenate([np.arange(TRI), np.arange(TRI)]).astype(np.int32)
    vsrc_np = np.pad(vsrc_np, (0, NTRI - 2 * TRI))
    pos_j = jnp.asarray(pos_np)
    vsrc_j = jnp.asarray(vsrc_np)

    dW3p = jnp.pad(dW3, ((0, 0), (0, TRIP - TRI)))
    db3p = jnp.pad(db3, (0, TRIP - TRI)).reshape(1, TRIP)

    degf = _deg_sc(dstp, onesd, zerosd)
    d0 = degf[:NP_, 0:1]
    d1 = degf[NP_:, 0:1]

    g1 = _gmat(xp, W1, d0, d1)
    parts1 = _msg_sc(g1, srcp, dstp, zerosh)
    g2 = _cb2(parts1[:NP_], parts1[NP_:], g1, d0, d1, b1.reshape(1, HH), W2)
    parts2 = _msg_sc(g2, srcp, dstp, zerosh)
    sums, cnts = _pool(parts2[:NP_], parts2[NP_:], g2, d0, d1,
                       b2.reshape(1, HH), batp)
    mu, lv, probsp = _head(sums, cnts, muW, mub.reshape(1, LL), lvW,
                           lvb.reshape(1, LL), eps, dW1, db1.reshape(1, HH),
                           dW2, db2.reshape(1, HH), dW3p, db3p)
    adjf = _triu_sc(probsp.reshape(BB * TRIP), pos_j, vsrc_j, zrow)
    adj = adjf.reshape(BB, ADJP)[:, :MM * MM].reshape(BB, MM, MM)
    return (adj, mu, lv)


# speedup vs baseline: 1.0420x; 1.0194x over previous
import functools

import numpy as np

import jax
import jax.numpy as jnp
from jax import lax
from jax.experimental import pallas as pl
from jax.experimental.pallas import tpu as pltpu
from jax.experimental.pallas import tpu_sc as plsc

NN = 10000
EE = 320000
DD = 128
HH = 128
LL = 64
BB = 64
MM = 100
TRI = MM * (MM - 1) // 2
TRIP = 4992

NP_ = 10240
RPT = NP_ // 16
CH = 128
EP = 327680
NCH = EP // (32 * CH)
PC = 40
FAST_P = 3
SLOW_P = 1
FAST_C = 1
DW = 128
ADJP = 10240
NTRI = 9904
RB = 512

_mesh = plsc.VectorSubcoreMesh(core_axis_name="c", subcore_axis_name="s")


@functools.partial(
    pl.kernel,
    out_type=jax.ShapeDtypeStruct((2 * 80, 128), jnp.float32),
    mesh=_mesh,
    scratch_types=[
        pltpu.VMEM((NCH, CH), jnp.int32),
        pltpu.VMEM((80, 128), jnp.float32),
        pltpu.VMEM((80,), jnp.int32),
        pltpu.VMEM_SHARED((80, 128), jnp.float32),
    ],
    compiler_params=pltpu.CompilerParams(needs_layout_passes=False),
)
def _deg_sc(dst_hbm, iota_hbm, zeros_hbm, out_hbm, idx_v, loc, iota_v, acc):
    c = lax.axis_index("c")
    s = lax.axis_index("s")
    wid = s * 2 + c
    pltpu.sync_copy(dst_hbm.at[pl.ds(wid * NCH, NCH)], idx_v)
    pltpu.sync_copy(iota_hbm, iota_v)

    @pl.when(s == 0)
    def _():
        pltpu.sync_copy(zeros_hbm, acc)

    def zbody(t, carry):
        loc[t // 8, pl.ds((t % 8) * 16, 16)] = jnp.zeros((16,), jnp.float32)
        return carry

    lax.fori_loop(0, 80 * 8, zbody, 0)

    ones16 = jnp.ones((16,), jnp.float32)

    def body(t, carry):
        idx = idx_v[t // 8, pl.ds((t % 8) * 16, 16)]
        plsc.addupdate_scatter(loc, [idx // 128, idx % 128], ones16)
        return carry

    lax.fori_loop(0, NCH * 8, body, 0)
    plsc.subcore_barrier()
    pltpu.sync_copy(loc, acc.at[iota_v], add=True)
    plsc.subcore_barrier()

    @pl.when(s < 10)
    def _():
        pltpu.sync_copy(acc.at[pl.ds(s * 8, 8)],
                        out_hbm.at[pl.ds(c * 80 + s * 8, 8)])


@functools.partial(
    pl.kernel,
    out_type=jax.ShapeDtypeStruct((2 * NP_, HH), jnp.float32),
    mesh=_mesh,
    scratch_types=[
        pltpu.VMEM((PC, CH), jnp.int32),
        pltpu.VMEM((PC, CH), jnp.int32),
        pltpu.VMEM((CH, HH), jnp.float32),
        pltpu.VMEM((CH, HH), jnp.float32),
        pltpu.VMEM_SHARED((NP_, HH), jnp.float32),
        pltpu.SemaphoreType.DMA,
        pltpu.SemaphoreType.DMA,
    ],
)
def _msg_sc(g_hbm, src_hbm, dst_hbm, zeros_hbm, out_hbm, sidx, didx,
            b0, b1, acc, s0, s1):
    c = lax.axis_index("c")
    s = lax.axis_index("s")
    bufs = (b0, b1)
    sems = (s0, s1)
    pltpu.sync_copy(zeros_hbm, acc.at[pl.ds(s * RPT, RPT)])
    plsc.subcore_barrier()

    def do_piece(base):
        pltpu.sync_copy(src_hbm.at[pl.ds(base, PC)], sidx)
        pltpu.sync_copy(dst_hbm.at[pl.ds(base, PC)], didx)
        pltpu.async_copy(g_hbm.at[sidx.at[0]], bufs[0], sems[0])

        def body(j2, carry):
            for k in range(2):
                j = j2 * 2 + k
                pltpu.make_async_copy(g_hbm.at[sidx.at[j]], bufs[k],
                                      sems[k]).wait()
                jn = j + 1

                @pl.when(jn < PC)
                def _():
                    pltpu.async_copy(g_hbm.at[sidx.at[jn]], bufs[(k + 1) % 2],
                                     sems[(k + 1) % 2])

                pltpu.sync_copy(bufs[k], acc.at[didx.at[j]], add=True)
            return carry

        lax.fori_loop(0, PC // 2, body, 0)

    @pl.when(c == FAST_C)
    def _():
        for p in range(FAST_P):
            do_piece((s * FAST_P + p) * PC)

    @pl.when(c != FAST_C)
    def _():
        for p in range(SLOW_P):
            do_piece(16 * FAST_P * PC + (s * SLOW_P + p) * PC)

    plsc.subcore_barrier()
    pltpu.sync_copy(acc.at[pl.ds(s * RPT, RPT)],
                    out_hbm.at[pl.ds(c * NP_ + s * RPT, RPT)])


@functools.partial(
    pl.kernel,
    out_type=jax.ShapeDtypeStruct((BB * ADJP,), jnp.float32),
    mesh=_mesh,
    scratch_types=[
        pltpu.VMEM((NTRI,), jnp.int32),
        pltpu.VMEM((NTRI,), jnp.int32),
        pltpu.VMEM((TRIP,), jnp.float32),
        pltpu.VMEM((ADJP,), jnp.float32),
    ],
    compiler_params=pltpu.CompilerParams(needs_layout_passes=False),
)
def _triu_sc(probs_hbm, pos_hbm, vsrc_hbm, zrow_hbm, out_hbm, pos_v, vsrc_v, prow, row):
    c = lax.axis_index("c")
    s = lax.axis_index("s")
    wid = s * 2 + c
    pltpu.sync_copy(pos_hbm, pos_v)
    pltpu.sync_copy(vsrc_hbm, vsrc_v)
    for gi in range(2):
        b = wid * 2 + gi
        pltpu.sync_copy(zrow_hbm, row)
        pltpu.sync_copy(probs_hbm.at[pl.ds(b * TRIP, TRIP)], prow)

        def body(t, carry):
            i16 = pos_v[pl.ds(t * 16, 16)]
            s16 = vsrc_v[pl.ds(t * 16, 16)]
            v = plsc.load_gather(prow, [s16])
            plsc.store_scatter(row, [i16], v)
            return carry

        lax.fori_loop(0, NTRI // 16, body, 0)
        pltpu.sync_copy(row, out_hbm.at[pl.ds(b * ADJP, ADJP)])


def _gmat_body(x_ref, w_ref, d0_ref, d1_ref, g_ref):
    dinv = lax.rsqrt(d0_ref[...] + d1_ref[...] + 1.0)
    g_ref[...] = jnp.dot(x_ref[...], w_ref[...],
                         preferred_element_type=jnp.float32) * dinv


_gmat = pl.pallas_call(
    _gmat_body,
    grid=(NP_ // RB,),
    in_specs=[
        pl.BlockSpec((RB, DD), lambda i: (i, 0)),
        pl.BlockSpec((DD, HH), lambda i: (0, 0)),
        pl.BlockSpec((RB, 1), lambda i: (i, 0)),
        pl.BlockSpec((RB, 1), lambda i: (i, 0)),
    ],
    out_specs=pl.BlockSpec((RB, HH), lambda i: (i, 0)),
    out_shape=jax.ShapeDtypeStruct((NP_, HH), jnp.float32),
)


def _cb2_body(p0_ref, p1_ref, g_ref, d0_ref, d1_ref, b_ref, w_ref, o_ref):
    i = pl.program_id(0)
    dinv = lax.rsqrt(d0_ref[...] + d1_ref[...] + 1.0)
    h = (p0_ref[...] + p1_ref[...] + g_ref[...]) * dinv + b_ref[...]
    h = jnp.maximum(h, 0.0)
    rows = i * RB + lax.broadcasted_iota(jnp.int32, (RB, 1), 0)
    h = jnp.where(rows < NN, h, 0.0)
    o_ref[...] = jnp.dot(h, w_ref[...],
                         preferred_element_type=jnp.float32) * dinv


_cb2 = pl.pallas_call(
    _cb2_body,
    grid=(NP_ // RB,),
    in_specs=[
        pl.BlockSpec((RB, HH), lambda i: (i, 0)),
        pl.BlockSpec((RB, HH), lambda i: (i, 0)),
        pl.BlockSpec((RB, HH), lambda i: (i, 0)),
        pl.BlockSpec((RB, 1), lambda i: (i, 0)),
        pl.BlockSpec((RB, 1), lambda i: (i, 0)),
        pl.BlockSpec((1, HH), lambda i: (0, 0)),
        pl.BlockSpec((HH, HH), lambda i: (0, 0)),
    ],
    out_specs=pl.BlockSpec((RB, HH), lambda i: (i, 0)),
    out_shape=jax.ShapeDtypeStruct((NP_, HH), jnp.float32),
)


def _pool_body(p0_ref, p1_ref, g_ref, d0_ref, d1_ref, b_ref, bat_ref,
               sums_ref, cnts_ref):
    i = pl.program_id(0)
    dinv = lax.rsqrt(d0_ref[...] + d1_ref[...] + 1.0)
    h = (p0_ref[...] + p1_ref[...] + g_ref[...]) * dinv + b_ref[...]
    h = jnp.maximum(h, 0.0)
    oh = (bat_ref[...] == lax.broadcasted_iota(jnp.int32, (RB, BB), 1))
    oh = oh.astype(jnp.float32)
    ps = lax.dot_general(oh, h, (((0,), (0,)), ((), ())),
                         preferred_element_type=jnp.float32)
    pc = lax.dot_general(oh, jnp.ones((RB, HH), jnp.float32),
                         (((0,), (0,)), ((), ())),
                         preferred_element_type=jnp.float32)

    @pl.when(i == 0)
    def _():
        sums_ref[...] = jnp.zeros_like(sums_ref)
        cnts_ref[...] = jnp.zeros_like(cnts_ref)

    sums_ref[...] += ps
    cnts_ref[...] += pc


_pool = pl.pallas_call(
    _pool_body,
    grid=(NP_ // RB,),
    in_specs=[
        pl.BlockSpec((RB, HH), lambda i: (i, 0)),
        pl.BlockSpec((RB, HH), lambda i: (i, 0)),
        pl.BlockSpec((RB, HH), lambda i: (i, 0)),
        pl.BlockSpec((RB, 1), lambda i: (i, 0)),
        pl.BlockSpec((RB, 1), lambda i: (i, 0)),
        pl.BlockSpec((1, HH), lambda i: (0, 0)),
        pl.BlockSpec((RB, 1), lambda i: (i, 0)),
    ],
    out_specs=[
        pl.BlockSpec((BB, HH), lambda i: (0, 0)),
        pl.BlockSpec((BB, HH), lambda i: (0, 0)),
    ],
    out_shape=[
        jax.ShapeDtypeStruct((BB, HH), jnp.float32),
        jax.ShapeDtypeStruct((BB, HH), jnp.float32),
    ],
)


def _head_body(sums_ref, cnts_ref, muW_ref, mub_ref, lvW_ref, lvb_ref, eps_ref,
               dW1_ref, db1_ref, dW2_ref, db2_ref, dW3_ref, db3_ref,
               mu_ref, lv_ref, probs_ref):
    hg = sums_ref[...] / jnp.maximum(cnts_ref[...], 1.0)
    mu = jnp.dot(hg, muW_ref[...], preferred_element_type=jnp.float32) + mub_ref[...]
    lv = jnp.dot(hg, lvW_ref[...], preferred_element_type=jnp.float32) + lvb_ref[...]
    mu_ref[...] = mu
    lv_ref[...] = lv
    z = mu + eps_ref[...] * jnp.exp(0.5 * lv)
    t1 = jnp.maximum(jnp.dot(z, dW1_ref[...],
                             preferred_element_type=jnp.float32) + db1_ref[...], 0.0)
    t2 = jnp.maximum(jnp.dot(t1, dW2_ref[...],
                             preferred_element_type=jnp.float32) + db2_ref[...], 0.0)
    logits = jnp.dot(t2, dW3_ref[...],
                     preferred_element_type=jnp.float32) + db3_ref[...]
    probs_ref[...] = jax.nn.sigmoid(logits)


_head = pl.pallas_call(
    _head_body,
    out_shape=[
        jax.ShapeDtypeStruct((BB, LL), jnp.float32),
        jax.ShapeDtypeStruct((BB, LL), jnp.float32),
        jax.ShapeDtypeStruct((BB, TRIP), jnp.float32),
    ],
)


def kernel(x, edge_index, batch, W1, b1, W2, b2, muW, mub, lvW, lvb,
           dW1, db1, dW2, db2, dW3, db3):
    f32 = jnp.float32
    src = edge_index[0].astype(jnp.int32)
    dst = edge_index[1].astype(jnp.int32)
    padi = jnp.full((EP - EE,), NN, jnp.int32)
    srcp = jnp.concatenate([src, padi]).reshape(EP // CH, CH)
    dstp = jnp.concatenate([dst, padi]).reshape(EP // CH, CH)
    xp = jnp.pad(x, ((0, NP_ - NN), (0, 0)))
    batp = jnp.pad(batch.astype(jnp.int32), (0, NP_ - NN),
                   constant_values=BB).reshape(NP_, 1)
    iota80 = jnp.arange(80, dtype=jnp.int32)
    zerosd = jnp.zeros((80, 128), f32)
    zerosh = jnp.zeros((RPT, HH), f32)
    zrow = jnp.zeros((ADJP,), f32)
    eps = jax.random.normal(jax.random.key(1), (BB, LL), dtype=f32)

    iu, ju = np.triu_indices(MM, k=1)
    pos_np = np.concatenate([iu * MM + ju, ju * MM + iu]).astype(np.int32)
    pos_np = np.pad(pos_np, (0, NTRI - 2 * TRI), constant_values=MM * MM + 8)
    vsrc_np = np.concatenate([np.arange(TRI), np.arange(TRI)]).astype(np.int32)
    vsrc_np = np.pad(vsrc_np, (0, NTRI - 2 * TRI))
    pos_j = jnp.asarray(pos_np)
    vsrc_j = jnp.asarray(vsrc_np)

    dW3p = jnp.pad(dW3, ((0, 0), (0, TRIP - TRI)))
    db3p = jnp.pad(db3, (0, TRIP - TRI)).reshape(1, TRIP)

    degf = _deg_sc(dstp, iota80, zerosd)
    d0 = degf[:80].reshape(NP_, 1)
    d1 = degf[80:].reshape(NP_, 1)

    g1 = _gmat(xp, W1, d0, d1)
    parts1 = _msg_sc(g1, srcp, dstp, zerosh)
    g2 = _cb2(parts1[:NP_], parts1[NP_:], g1, d0, d1, b1.reshape(1, HH), W2)
    parts2 = _msg_sc(g2, srcp, dstp, zerosh)
    sums, cnts = _pool(parts2[:NP_], parts2[NP_:], g2, d0, d1,
                       b2.reshape(1, HH), batp)
    mu, lv, probsp = _head(sums, cnts, muW, mub.reshape(1, LL), lvW,
                           lvb.reshape(1, LL), eps, dW1, db1.reshape(1, HH),
                           dW2, db2.reshape(1, HH), dW3p, db3p)
    adjf = _triu_sc(probsp.reshape(BB * TRIP), pos_j, vsrc_j, zrow)
    adj = adjf.reshape(BB, ADJP)[:, :MM * MM].reshape(BB, MM, MM)
    return (adj, mu, lv)
